# baseline (speedup 1.0000x reference)
import jax
import jax.numpy as jnp
from jax import lax
from jax.experimental import pallas as pl
from jax.experimental.pallas import tpu as pltpu

N = 10000
T = 12
D = 128
H = 128
E = 320000

BLK = 1000
NB = N // BLK
SBLK = 8000

_SQRT1_2 = 0.7071067811865476


def _gelu(x):
    return 0.5 * x * (1.0 + lax.erf(x * _SQRT1_2))


def _bnstats_body(x_ref, out_ref):
    xb = x_ref[...]
    st = jnp.concatenate([jnp.sum(xb, 0, keepdims=True),
                          jnp.sum(xb * xb, 0, keepdims=True)], 0)
    i = pl.program_id(0)

    @pl.when(i == 0)
    def _():
        out_ref[...] = st

    @pl.when(i > 0)
    def _():
        out_ref[...] += st


def _bnstats(x):
    return pl.pallas_call(
        _bnstats_body,
        out_shape=jax.ShapeDtypeStruct((2, D), jnp.float32),
        grid=((N * T) // SBLK,),
        in_specs=[pl.BlockSpec((SBLK, D), lambda i: (i, 0))],
        out_specs=pl.BlockSpec((2, D), lambda i: (0, 0)),
    )(x)


def _norms_body(dso_ref, ddi_ref, ns_ref, nd_ref):
    def nrm(dref):
        d = dref[0][:, 0:1]
        v = jnp.where(d > 0, lax.rsqrt(jnp.maximum(d, 1e-12)), 0.0)
        return jnp.broadcast_to(v, (BLK, D))
    ns_ref[...] = nrm(dso_ref)
    nd_ref[...] = nrm(ddi_ref)


def _norms(degs):
    return pl.pallas_call(
        _norms_body,
        out_shape=(jax.ShapeDtypeStruct((N, D), jnp.float32),
                   jax.ShapeDtypeStruct((N, D), jnp.float32)),
        grid=(NB,),
        in_specs=[pl.BlockSpec((1, BLK, 16), lambda j: (0, j, 0)),
                  pl.BlockSpec((1, BLK, 16), lambda j: (1, j, 0))],
        out_specs=(pl.BlockSpec((BLK, D), lambda j: (j, 0)),
                   pl.BlockSpec((BLK, D), lambda j: (j, 0))),
    )(degs, degs)


def _prep_body(h_ref, sums_ref, gamma_ref, beta_ref, ns_ref, hbt_ref, x1_ref):
    cnt = 1.0 / (N * T)
    mean = sums_ref[0:1, :] * cnt
    var = sums_ref[1:2, :] * cnt - mean * mean
    rstd = lax.rsqrt(var + 1e-5)
    ns = ns_ref[...]
    for t in range(T):
        xn = ((h_ref[:, t, :] - mean) * rstd * gamma_ref[...]
              + beta_ref[...])
        hbt_ref[t] = xn
        x1_ref[:, t, :] = xn * ns


def _prep(h, sums, gamma, beta, ns_tab):
    return pl.pallas_call(
        _prep_body,
        out_shape=(jax.ShapeDtypeStruct((T, N, D), jnp.float32),
                   jax.ShapeDtypeStruct((N, T, D), jnp.float32)),
        grid=(NB,),
        in_specs=[
            pl.BlockSpec((BLK, T, D), lambda j: (j, 0, 0)),
            pl.BlockSpec((2, D), lambda j: (0, 0)),
            pl.BlockSpec((1, D), lambda j: (0, 0)),
            pl.BlockSpec((1, D), lambda j: (0, 0)),
            pl.BlockSpec((BLK, D), lambda j: (j, 0)),
        ],
        out_specs=(pl.BlockSpec((T, BLK, D), lambda j: (0, j, 0)),
                   pl.BlockSpec((BLK, T, D), lambda j: (j, 0, 0))),
    )(h, sums, gamma, beta, ns_tab)


def _dense1_body(agg_ref, ns_ref, nd_ref, w_ref, b_ref, out_ref):
    ns = ns_ref[...]
    nd = nd_ref[...]
    w = w_ref[...]
    b = b_ref[...]
    for t in range(T):
        a = agg_ref[:, t, :] * nd
        y = jnp.dot(a, w, preferred_element_type=jnp.float32) + b
        out_ref[:, t, :] = _gelu(y) * ns


def _dense1(agg, ns_tab, nd_tab, W, b):
    return pl.pallas_call(
        _dense1_body,
        out_shape=jax.ShapeDtypeStruct((N, T, D), jnp.float32),
        grid=(NB,),
        in_specs=[
            pl.BlockSpec((BLK, T, D), lambda j: (j, 0, 0)),
            pl.BlockSpec((BLK, D), lambda j: (j, 0)),
            pl.BlockSpec((BLK, D), lambda j: (j, 0)),
            pl.BlockSpec((D, H), lambda j: (0, 0)),
            pl.BlockSpec((1, H), lambda j: (0, 0)),
        ],
        out_specs=pl.BlockSpec((BLK, T, D), lambda j: (j, 0, 0)),
    )(agg, ns_tab, nd_tab, W, b)


def _dense2_body(agg_ref, nd_ref, w_ref, b_ref, out_ref):
    nd = nd_ref[...]
    w = w_ref[...]
    b = b_ref[...]
    j = pl.program_id(0)
    for t in range(T):
        a = agg_ref[:, t, :] * nd
        y = jnp.dot(a, w, preferred_element_type=jnp.float32) + b
        s = jnp.sum(_gelu(y), 0, keepdims=True) * (1.0 / N)
        sb = jnp.broadcast_to(s, (8, H))

        @pl.when(j == 0)
        def _():
            out_ref[t] = sb

        @pl.when(j > 0)
        def _():
            out_ref[t] += sb


def _dense2(agg, nd_tab, W, b):
    return pl.pallas_call(
        _dense2_body,
        out_shape=jax.ShapeDtypeStruct((T, 8, H), jnp.float32),
        grid=(NB,),
        in_specs=[
            pl.BlockSpec((BLK, T, D), lambda j: (j, 0, 0)),
            pl.BlockSpec((BLK, D), lambda j: (j, 0)),
            pl.BlockSpec((D, H), lambda j: (0, 0)),
            pl.BlockSpec((1, H), lambda j: (0, 0)),
        ],
        out_specs=pl.BlockSpec((T, 8, H), lambda j: (0, 0, 0)),
    )(agg, nd_tab, W, b)


LBLK = 2000
LNB = N // LBLK


def _lstm_body(hbt_ref, wih_ref, whh_ref, bih_ref, bhh_ref, out_ref,
               hst, cst, osum):
    t = pl.program_id(0)
    j = pl.program_id(1)
    rows = pl.ds(j * LBLK, LBLK)

    @pl.when(t == 0)
    def _():
        hst[rows, :] = jnp.zeros((LBLK, H), jnp.float32)
        cst[rows, :] = jnp.zeros((LBLK, H), jnp.float32)

    x = hbt_ref[0]
    hp = hst[rows, :]
    cp = cst[rows, :]
    gates = (jnp.dot(x, wih_ref[...], preferred_element_type=jnp.float32)
             + jnp.dot(hp, whh_ref[...], preferred_element_type=jnp.float32)
             + bih_ref[...] + bhh_ref[...])
    ig = jax.nn.sigmoid(gates[:, 0:H])
    fg = jax.nn.sigmoid(gates[:, H:2 * H])
    gg = jnp.tanh(gates[:, 2 * H:3 * H])
    og = jax.nn.sigmoid(gates[:, 3 * H:4 * H])
    c = fg * cp + ig * gg
    hh = og * jnp.tanh(c)
    hst[rows, :] = hh
    cst[rows, :] = c

    @pl.when(t == 0)
    def _():
        osum[rows, :] = hh

    @pl.when(t > 0)
    def _():
        osum[rows, :] += hh

    @pl.when(t == T - 1)
    def _():
        out_ref[...] = osum[rows, :] * (1.0 / T)


def _lstm(hbt, wihT, whhT, bih, bhh):
    return pl.pallas_call(
        _lstm_body,
        out_shape=jax.ShapeDtypeStruct((N, H), jnp.float32),
        grid=(T, LNB),
        in_specs=[
            pl.BlockSpec((1, LBLK, D), lambda t, j: (t, j, 0)),
            pl.BlockSpec((D, 4 * H), lambda t, j: (0, 0)),
            pl.BlockSpec((H, 4 * H), lambda t, j: (0, 0)),
            pl.BlockSpec((1, 4 * H), lambda t, j: (0, 0)),
            pl.BlockSpec((1, 4 * H), lambda t, j: (0, 0)),
        ],
        out_specs=pl.BlockSpec((LBLK, H), lambda t, j: (j, 0)),
        scratch_shapes=[
            pltpu.VMEM((N, H), jnp.float32),
            pltpu.VMEM((N, H), jnp.float32),
            pltpu.VMEM((N, H), jnp.float32),
        ],
    )(hbt, wihT, whhT, bih, bhh)


def kernel(h, edge_index, gamma, beta, W1, b1, W2, b2, W_ih, W_hh, b_ih, b_hh):
    src = edge_index[0]
    dst = edge_index[1]
    deg_out = jnp.zeros((N,), jnp.float32).at[src].add(1.0)
    deg_in = jnp.zeros((N,), jnp.float32).at[dst].add(1.0)
    degs = jnp.broadcast_to(
        jnp.stack([deg_out, deg_in])[:, :, None], (2, N, 16))
    ns_tab, nd_tab = _norms(degs)
    sums = _bnstats(h.reshape(N * T, D))
    hbt, x1 = _prep(h, sums, gamma.reshape(1, D), beta.reshape(1, D), ns_tab)

    def agg_all(x_ntd):
        m = x_ntd.reshape(N, T * D)[src]
        s = jax.ops.segment_sum(m, dst, num_segments=N)
        return s.reshape(N, T, D)

    agg1 = agg_all(x1)
    x2 = _dense1(agg1, ns_tab, nd_tab, W1, b1.reshape(1, H))
    agg2 = agg_all(x2)
    hs = _dense2(agg2, nd_tab, W2, b2.reshape(1, H))[:, 0, :]
    ht = _lstm(hbt, W_ih.T, W_hh.T, b_ih.reshape(1, 4 * H),
               b_hh.reshape(1, 4 * H))
    return (hs.reshape(1, T, H), ht)


# speedup vs baseline: 1.1031x; 1.1031x over previous
import jax
import jax.numpy as jnp
from jax import lax
from jax.experimental import pallas as pl
from jax.experimental.pallas import tpu as pltpu

N = 10000
T = 12
D = 128
H = 128
E = 320000

BLK = 1000
NB = N // BLK
SBLK = 8000

_SQRT1_2 = 0.7071067811865476


def _gelu(x):
    return 0.5 * x * (1.0 + lax.erf(x * _SQRT1_2))


def _bnstats_body(x_ref, out_ref):
    xb = x_ref[...]
    st = jnp.concatenate([jnp.sum(xb, 0, keepdims=True),
                          jnp.sum(xb * xb, 0, keepdims=True)], 0)
    i = pl.program_id(0)

    @pl.when(i == 0)
    def _():
        out_ref[...] = st

    @pl.when(i > 0)
    def _():
        out_ref[...] += st


def _bnstats(x):
    return pl.pallas_call(
        _bnstats_body,
        out_shape=jax.ShapeDtypeStruct((2, D), jnp.float32),
        grid=((N * T) // SBLK,),
        in_specs=[pl.BlockSpec((SBLK, D), lambda i: (i, 0))],
        out_specs=pl.BlockSpec((2, D), lambda i: (0, 0)),
    )(x)


def _norms_body(dso_ref, ddi_ref, ns_ref, nd_ref):
    def nrm(dref):
        d = dref[0][:, 0:1]
        v = jnp.where(d > 0, lax.rsqrt(jnp.maximum(d, 1e-12)), 0.0)
        return jnp.broadcast_to(v, (BLK, D))
    ns_ref[...] = nrm(dso_ref)
    nd_ref[...] = nrm(ddi_ref)


def _norms(degs):
    return pl.pallas_call(
        _norms_body,
        out_shape=(jax.ShapeDtypeStruct((N, D), jnp.float32),
                   jax.ShapeDtypeStruct((N, D), jnp.float32)),
        grid=(NB,),
        in_specs=[pl.BlockSpec((1, BLK, 16), lambda j: (0, j, 0)),
                  pl.BlockSpec((1, BLK, 16), lambda j: (1, j, 0))],
        out_specs=(pl.BlockSpec((BLK, D), lambda j: (j, 0)),
                   pl.BlockSpec((BLK, D), lambda j: (j, 0))),
    )(degs, degs)


def _prep_body(h_ref, sums_ref, gamma_ref, beta_ref, ns_ref, hbt_ref, x1_ref):
    cnt = 1.0 / (N * T)
    mean = sums_ref[0:1, :] * cnt
    var = sums_ref[1:2, :] * cnt - mean * mean
    rstd = lax.rsqrt(var + 1e-5)
    ns = ns_ref[...]
    for t in range(T):
        xn = ((h_ref[:, t, :] - mean) * rstd * gamma_ref[...]
              + beta_ref[...])
        hbt_ref[t] = xn
        x1_ref[:, t, :] = xn * ns


def _prep(h, sums, gamma, beta, ns_tab):
    return pl.pallas_call(
        _prep_body,
        out_shape=(jax.ShapeDtypeStruct((T, N, D), jnp.float32),
                   jax.ShapeDtypeStruct((N, T, D), jnp.float32)),
        grid=(NB,),
        in_specs=[
            pl.BlockSpec((BLK, T, D), lambda j: (j, 0, 0)),
            pl.BlockSpec((2, D), lambda j: (0, 0)),
            pl.BlockSpec((1, D), lambda j: (0, 0)),
            pl.BlockSpec((1, D), lambda j: (0, 0)),
            pl.BlockSpec((BLK, D), lambda j: (j, 0)),
        ],
        out_specs=(pl.BlockSpec((T, BLK, D), lambda j: (0, j, 0)),
                   pl.BlockSpec((BLK, T, D), lambda j: (j, 0, 0))),
    )(h, sums, gamma, beta, ns_tab)


def _dense1_body(agg_ref, ns_ref, nd_ref, w_ref, b_ref, out_ref):
    ns = ns_ref[...]
    nd = nd_ref[...]
    w = w_ref[...]
    b = b_ref[...]
    for t in range(T):
        a = agg_ref[:, t, :] * nd
        y = jnp.dot(a, w, preferred_element_type=jnp.float32) + b
        out_ref[:, t, :] = _gelu(y) * ns


def _dense1(agg, ns_tab, nd_tab, W, b):
    return pl.pallas_call(
        _dense1_body,
        out_shape=jax.ShapeDtypeStruct((N, T, D), jnp.float32),
        grid=(NB,),
        in_specs=[
            pl.BlockSpec((BLK, T, D), lambda j: (j, 0, 0)),
            pl.BlockSpec((BLK, D), lambda j: (j, 0)),
            pl.BlockSpec((BLK, D), lambda j: (j, 0)),
            pl.BlockSpec((D, H), lambda j: (0, 0)),
            pl.BlockSpec((1, H), lambda j: (0, 0)),
        ],
        out_specs=pl.BlockSpec((BLK, T, D), lambda j: (j, 0, 0)),
    )(agg, ns_tab, nd_tab, W, b)


def _dense2_body(agg_ref, nd_ref, w_ref, b_ref, out_ref):
    nd = nd_ref[...]
    w = w_ref[...]
    b = b_ref[...]
    j = pl.program_id(0)
    for t in range(T):
        a = agg_ref[:, t, :] * nd
        y = jnp.dot(a, w, preferred_element_type=jnp.float32) + b
        s = jnp.sum(_gelu(y), 0, keepdims=True) * (1.0 / N)
        sb = jnp.broadcast_to(s, (8, H))

        @pl.when(j == 0)
        def _():
            out_ref[t] = sb

        @pl.when(j > 0)
        def _():
            out_ref[t] += sb


def _dense2(agg, nd_tab, W, b):
    return pl.pallas_call(
        _dense2_body,
        out_shape=jax.ShapeDtypeStruct((T, 8, H), jnp.float32),
        grid=(NB,),
        in_specs=[
            pl.BlockSpec((BLK, T, D), lambda j: (j, 0, 0)),
            pl.BlockSpec((BLK, D), lambda j: (j, 0)),
            pl.BlockSpec((D, H), lambda j: (0, 0)),
            pl.BlockSpec((1, H), lambda j: (0, 0)),
        ],
        out_specs=pl.BlockSpec((T, 8, H), lambda j: (0, 0, 0)),
    )(agg, nd_tab, W, b)


LBLK = 2000
LNB = N // LBLK


def _lstm_body(hbt_ref, wih_ref, whh_ref, bih_ref, bhh_ref, out_ref,
               hst, cst, osum):
    t = pl.program_id(0)
    j = pl.program_id(1)
    rows = pl.ds(j * LBLK, LBLK)

    @pl.when(t == 0)
    def _():
        hst[rows, :] = jnp.zeros((LBLK, H), jnp.float32)
        cst[rows, :] = jnp.zeros((LBLK, H), jnp.float32)

    x = hbt_ref[0]
    hp = hst[rows, :]
    cp = cst[rows, :]
    gates = (jnp.dot(x, wih_ref[...], preferred_element_type=jnp.float32)
             + jnp.dot(hp, whh_ref[...], preferred_element_type=jnp.float32)
             + bih_ref[...] + bhh_ref[...])
    ig = jax.nn.sigmoid(gates[:, 0:H])
    fg = jax.nn.sigmoid(gates[:, H:2 * H])
    gg = jnp.tanh(gates[:, 2 * H:3 * H])
    og = jax.nn.sigmoid(gates[:, 3 * H:4 * H])
    c = fg * cp + ig * gg
    hh = og * jnp.tanh(c)
    hst[rows, :] = hh
    cst[rows, :] = c

    @pl.when(t == 0)
    def _():
        osum[rows, :] = hh

    @pl.when(t > 0)
    def _():
        osum[rows, :] += hh

    @pl.when(t == T - 1)
    def _():
        out_ref[...] = osum[rows, :] * (1.0 / T)


def _lstm(hbt, wihT, whhT, bih, bhh):
    return pl.pallas_call(
        _lstm_body,
        out_shape=jax.ShapeDtypeStruct((N, H), jnp.float32),
        grid=(T, LNB),
        in_specs=[
            pl.BlockSpec((1, LBLK, D), lambda t, j: (t, j, 0)),
            pl.BlockSpec((D, 4 * H), lambda t, j: (0, 0)),
            pl.BlockSpec((H, 4 * H), lambda t, j: (0, 0)),
            pl.BlockSpec((1, 4 * H), lambda t, j: (0, 0)),
            pl.BlockSpec((1, 4 * H), lambda t, j: (0, 0)),
        ],
        out_specs=pl.BlockSpec((LBLK, H), lambda t, j: (j, 0)),
        scratch_shapes=[
            pltpu.VMEM((N, H), jnp.float32),
            pltpu.VMEM((N, H), jnp.float32),
            pltpu.VMEM((N, H), jnp.float32),
        ],
    )(hbt, wihT, whhT, bih, bhh)


def kernel(h, edge_index, gamma, beta, W1, b1, W2, b2, W_ih, W_hh, b_ih, b_hh):
    src = edge_index[0]
    dst = edge_index[1]
    deg_out = jnp.zeros((N,), jnp.float32).at[src].add(1.0)
    deg_in = jnp.zeros((N,), jnp.float32).at[dst].add(1.0)
    degs = jnp.broadcast_to(
        jnp.stack([deg_out, deg_in])[:, :, None], (2, N, 16))
    ns_tab, nd_tab = _norms(degs)
    sums = _bnstats(h.reshape(N * T, D))
    hbt, x1 = _prep(h, sums, gamma.reshape(1, D), beta.reshape(1, D), ns_tab)

    order = jnp.argsort(dst)
    ssrc = src[order]
    sdst = dst[order]

    def agg_all(x_ntd):
        m = x_ntd.reshape(N, T * D)[ssrc]
        s = jax.ops.segment_sum(
            m, sdst, num_segments=N, indices_are_sorted=True)
        return s.reshape(N, T, D)

    agg1 = agg_all(x1)
    x2 = _dense1(agg1, ns_tab, nd_tab, W1, b1.reshape(1, H))
    agg2 = agg_all(x2)
    hs = _dense2(agg2, nd_tab, W2, b2.reshape(1, H))[:, 0, :]
    ht = _lstm(hbt, W_ih.T, W_hh.T, b_ih.reshape(1, 4 * H),
               b_hh.reshape(1, 4 * H))
    return (hs.reshape(1, T, H), ht)
